# baseline (speedup 1.0000x reference)
import functools

import jax
import jax.numpy as jnp
import numpy as np
from jax import lax
from jax.experimental import pallas as pl
from jax.experimental.pallas import tpu as pltpu
from jax.experimental.pallas import tpu_sc as plsc

VOCAB = 100000
EMBED_DIM = 128
BATCH = 1024
SEQ = 200

_info = plsc.get_sparse_core_info()
NC, NS, L = _info.num_cores, _info.num_subcores, _info.num_lanes
NW = NC * NS
SC_BATCH = 896
ROWS = SC_BATCH * SEQ
ROWS_PER_W = ROWS // NW
CHUNK = 160
NBUF = 5
NCH = ROWS_PER_W // CHUNK
NGRP = NCH // NBUF
GLA = 3


def _positional_encoding_np(position, d_model):
    angle_rates = 1 / np.power(
        10000, 2 * (np.arange(d_model)[np.newaxis, :] // 2) / np.float32(d_model)
    )
    angle_rads = np.arange(position)[:, np.newaxis] * angle_rates
    angle_rads[:, 0::2] = np.sin(angle_rads[:, 0::2])
    angle_rads[:, 1::2] = np.cos(angle_rads[:, 1::2])
    return angle_rads.astype(np.float32)


_POS = _positional_encoding_np(SEQ, EMBED_DIM)


def _body(idx_hbm, table_hbm, pos_hbm, out_hbm, pos_v, *bufs):
    rows = bufs[0:NBUF]
    idxb = bufs[NBUF : 2 * NBUF]
    gsem = bufs[2 * NBUF : 3 * NBUF]
    wsem = bufs[3 * NBUF : 4 * NBUF]
    isem = bufs[4 * NBUF : 5 * NBUF]
    wid = lax.axis_index("s") * NC + lax.axis_index("c")
    wbase = wid * ROWS_PER_W
    pltpu.sync_copy(pos_hbm, pos_v)

    def i_desc(c, b):
        return pltpu.make_async_copy(
            idx_hbm.at[pl.ds(wbase + c * CHUNK, CHUNK)], idxb[b], isem[b]
        )

    def g_desc(b):
        return pltpu.make_async_copy(table_hbm.at[idxb[b]], rows[b], gsem[b])

    def w_desc(c, b):
        return pltpu.make_async_copy(
            rows[b], out_hbm.at[pl.ds(wbase + c * CHUNK, CHUNK)], wsem[b]
        )

    def add_chunk(c, b):
        buf = rows[b]
        off = lax.rem(c * CHUNK, SEQ)

        @plsc.parallel_loop(0, CHUNK, step=1, unroll=4)
        def _(r):
            pr = r + off
            pr = jnp.where(pr >= SEQ, pr - SEQ, pr)
            for cc in range(EMBED_DIM // L):
                sl = pl.ds(cc * L, L)
                buf[r, sl] = buf[r, sl] + pos_v[pr, sl]

    def chunk_step(c, b, pref_wait, pref, pref_i=True):
        if pref:
            bg = (b + GLA) % NBUF
            if pref_wait:
                w_desc(c - (NBUF - GLA), bg).wait()
            i_desc(c + GLA, bg).wait()
            g_desc(bg).start()
            if pref_i:
                i_desc(c + GLA + 1, (b + GLA + 1) % NBUF).start()
        g_desc(b).wait()
        add_chunk(c, b)
        w_desc(c, b).start()

    for b in range(GLA + 1):
        i_desc(b, b).start()
    for b in range(GLA):
        i_desc(b, b).wait()
        g_desc(b).start()
    for b in range(NBUF):
        chunk_step(b, b, pref_wait=(b >= NBUF - GLA), pref=True)

    def outer(t, _):
        c0 = t * NBUF
        for b in range(NBUF):
            chunk_step(c0 + b, b, pref_wait=True, pref=True)
        return 0

    lax.fori_loop(1, NGRP - 1, outer, 0)

    cL = (NGRP - 1) * NBUF
    for b in range(NBUF):
        c = cL + b
        chunk_step(
            c, b, pref_wait=True, pref=(c + GLA < NCH), pref_i=(c + GLA + 1 < NCH)
        )
    for b in range(NBUF):
        w_desc(cL + b, b).wait()


@functools.partial(jax.jit, static_argnames=())
def kernel(x, table):
    idx_flat = x[:SC_BATCH].reshape(-1)
    pos = jnp.asarray(_POS)
    mesh = plsc.VectorSubcoreMesh(core_axis_name="c", subcore_axis_name="s")
    k = functools.partial(
        pl.kernel,
        mesh=mesh,
        out_type=jax.ShapeDtypeStruct((ROWS, EMBED_DIM), jnp.float32),
        scratch_types=(
            [pltpu.VMEM((SEQ, EMBED_DIM), jnp.float32)]
            + [pltpu.VMEM((CHUNK, EMBED_DIM), jnp.float32) for _ in range(NBUF)]
            + [pltpu.VMEM((CHUNK,), jnp.int32) for _ in range(NBUF)]
            + [pltpu.SemaphoreType.DMA for _ in range(3 * NBUF)]
        ),
    )(_body)
    out_flat = k(idx_flat, table, pos)
    out_sc = out_flat.reshape(SC_BATCH, SEQ, EMBED_DIM)
    out_tc = jnp.take(table, x[SC_BATCH:], axis=0) + pos[None]
    return jnp.concatenate([out_sc, out_tc], axis=0)


# speedup vs baseline: 1.7210x; 1.7210x over previous
import functools

import jax
import jax.numpy as jnp
import numpy as np
from jax import lax
from jax.experimental import pallas as pl
from jax.experimental.pallas import tpu as pltpu
from jax.experimental.pallas import tpu_sc as plsc

VOCAB = 100000
EMBED_DIM = 128
BATCH = 1024
SEQ = 200

_info = plsc.get_sparse_core_info()
NC, NS, L = _info.num_cores, _info.num_subcores, _info.num_lanes
NW = NC * NS
ROWS = BATCH * SEQ
ROWS_PER_W = ROWS // NW
CHUNK = 160
NBUF = 4
NCH = ROWS_PER_W // CHUNK
NGRP = NCH // NBUF


def _positional_encoding_np(position, d_model):
    angle_rates = 1 / np.power(
        10000, 2 * (np.arange(d_model)[np.newaxis, :] // 2) / np.float32(d_model)
    )
    angle_rads = np.arange(position)[:, np.newaxis] * angle_rates
    angle_rads[:, 0::2] = np.sin(angle_rads[:, 0::2])
    angle_rads[:, 1::2] = np.cos(angle_rads[:, 1::2])
    return angle_rads.astype(np.float32)


_POS = _positional_encoding_np(SEQ, EMBED_DIM)


def _body(idx_hbm, table_hbm, pos_hbm, out_hbm, pos_v, idx_v, *bufs):
    rows = bufs[0:NBUF]
    gsem = bufs[NBUF : 2 * NBUF]
    wsem = bufs[2 * NBUF : 3 * NBUF]
    wid = lax.axis_index("s") * NC + lax.axis_index("c")
    wbase = wid * ROWS_PER_W
    pltpu.sync_copy(pos_hbm, pos_v)
    pltpu.sync_copy(idx_hbm.at[pl.ds(wbase, ROWS_PER_W)], idx_v)

    def g_desc(c, b):
        return pltpu.make_async_copy(
            table_hbm.at[idx_v.at[pl.ds(c * CHUNK, CHUNK)]], rows[b], gsem[b]
        )

    def w_desc(c, b):
        return pltpu.make_async_copy(
            rows[b], out_hbm.at[pl.ds(wbase + c * CHUNK, CHUNK)], wsem[b]
        )

    def add_chunk(c, b):
        buf = rows[b]
        off = lax.rem(c * CHUNK, SEQ)

        @plsc.parallel_loop(0, CHUNK, step=1, unroll=4)
        def _(r):
            pr = r + off
            pr = jnp.where(pr >= SEQ, pr - SEQ, pr)
            for cc in range(EMBED_DIM // L):
                sl = pl.ds(cc * L, L)
                buf[r, sl] = buf[r, sl] + pos_v[pr, sl]

    def chunk_step(c, b, pref_wait, pref):
        if pref:
            b2 = (b + 2) % NBUF
            if pref_wait:
                w_desc(c - 2, b2).wait()
            g_desc(c + 2, b2).start()
        g_desc(c, b).wait()
        add_chunk(c, b)
        w_desc(c, b).start()

    g_desc(0, 0).start()
    g_desc(1, 1).start()
    for b in range(NBUF):
        chunk_step(b, b, pref_wait=(b >= 2), pref=True)

    def outer(t, _):
        c0 = t * NBUF
        for b in range(NBUF):
            chunk_step(c0 + b, b, pref_wait=True, pref=True)
        return 0

    lax.fori_loop(1, NGRP - 1, outer, 0)

    cL = (NGRP - 1) * NBUF
    for b in range(NBUF):
        chunk_step(cL + b, b, pref_wait=True, pref=(cL + b + 2 < NCH))
    for b in range(NBUF):
        w_desc(cL + b, b).wait()


@functools.partial(jax.jit, static_argnames=())
def kernel(x, table):
    idx_flat = x.reshape(-1)
    pos = jnp.asarray(_POS)
    mesh = plsc.VectorSubcoreMesh(core_axis_name="c", subcore_axis_name="s")
    k = functools.partial(
        pl.kernel,
        mesh=mesh,
        out_type=jax.ShapeDtypeStruct((ROWS, EMBED_DIM), jnp.float32),
        scratch_types=(
            [
                pltpu.VMEM((SEQ, EMBED_DIM), jnp.float32),
                pltpu.VMEM((ROWS_PER_W,), jnp.int32),
            ]
            + [pltpu.VMEM((CHUNK, EMBED_DIM), jnp.float32) for _ in range(NBUF)]
            + [pltpu.SemaphoreType.DMA for _ in range(2 * NBUF)]
        ),
    )(_body)
    out_flat = k(idx_flat, table, pos)
    return out_flat.reshape(BATCH, SEQ, EMBED_DIM)
